# baseline (speedup 1.0000x reference)
import functools

import jax
import jax.numpy as jnp
from jax import lax
from jax.experimental import pallas as pl
from jax.experimental.pallas import tpu as pltpu
from jax.experimental.pallas import tpu_sc as plsc

_N_TOKENS = 32768
_D_MODEL = 4096
_NUM_EXPERTS = 64
_TOP_K = 8
_MM_BLOCK = 1024

_NC = 2
_NS = 16
_NW = _NC * _NS
_LANES = 16
_CHUNK = 256
_ROWS_PER_W = _N_TOKENS // _NW

_NEG_INF = float("-inf")


def _matmul_block(x_ref, w_ref, b_ref, out_ref):
    out_ref[...] = lax.dot_general(
        x_ref[...], w_ref[...], (((1,), (1,)), ((), ())),
        preferred_element_type=jnp.float32,
    ) + b_ref[...]


def _router_logits(x, W, b):
    grid = (x.shape[0] // _MM_BLOCK,)
    return pl.pallas_call(
        _matmul_block,
        grid=grid,
        in_specs=[
            pl.BlockSpec((_MM_BLOCK, _D_MODEL), lambda i: (i, 0)),
            pl.BlockSpec((_NUM_EXPERTS, _D_MODEL), lambda i: (0, 0)),
            pl.BlockSpec((1, _NUM_EXPERTS), lambda i: (0, 0)),
        ],
        out_specs=pl.BlockSpec((_MM_BLOCK, _NUM_EXPERTS), lambda i: (i, 0)),
        out_shape=jax.ShapeDtypeStruct((x.shape[0], _NUM_EXPERTS), jnp.float32),
        compiler_params=pltpu.CompilerParams(
            dimension_semantics=("parallel",),
        ),
    )(x, W, b.reshape(1, _NUM_EXPERTS))


def _topk_softmax_subblock(sb, lbuf, wbuf, ibuf):
    lanes = lax.iota(jnp.int32, _LANES)
    lrow0 = lanes * _NUM_EXPERTS + jnp.int32(sb * _LANES * _NUM_EXPERTS)

    def estep(e, carry):
        t = list(carry[:_TOP_K])
        ti = list(carry[_TOP_K:])
        v = plsc.load_gather(lbuf, [lrow0 + e])
        vi = jnp.full((_LANES,), e, jnp.int32)
        for j in range(_TOP_K):
            c = v > t[j]
            t[j], v = jnp.where(c, v, t[j]), jnp.where(c, t[j], v)
            ti[j], vi = jnp.where(c, vi, ti[j]), jnp.where(c, ti[j], vi)
        return tuple(t) + tuple(ti)

    init = (tuple(jnp.full((_LANES,), _NEG_INF, jnp.float32) for _ in range(_TOP_K))
            + tuple(jnp.zeros((_LANES,), jnp.int32) for _ in range(_TOP_K)))
    carry = lax.fori_loop(0, _NUM_EXPERTS, estep, init)
    t = carry[:_TOP_K]
    ti = carry[_TOP_K:]

    ew = [jnp.exp(t[j] - t[0]) for j in range(_TOP_K)]
    denom = ew[0]
    for j in range(1, _TOP_K):
        denom = denom + ew[j]
    inv = jnp.float32(1.0) / denom

    zero = jnp.zeros((_LANES,), jnp.float32)
    base = sb * _LANES * _NUM_EXPERTS
    for o in range(_LANES * _NUM_EXPERTS // _LANES):
        wbuf[pl.ds(base + o * _LANES, _LANES)] = zero
    for j in range(_TOP_K):
        plsc.store_scatter(wbuf, [lrow0 + ti[j]], ew[j] * inv)
        plsc.store_scatter(
            ibuf,
            [lanes * _TOP_K + jnp.int32(sb * _LANES * _TOP_K + j)],
            ti[j])


def _gating_sc(logits_flat):
    mesh = plsc.VectorSubcoreMesh(core_axis_name="c", subcore_axis_name="s")

    @functools.partial(
        pl.kernel,
        out_type=[
            jax.ShapeDtypeStruct((_N_TOKENS * _NUM_EXPERTS,), jnp.float32),
            jax.ShapeDtypeStruct((_N_TOKENS * _TOP_K,), jnp.int32),
        ],
        mesh=mesh,
        scratch_types=[
            pltpu.VMEM((_CHUNK * _NUM_EXPERTS,), jnp.float32),
            pltpu.VMEM((_CHUNK * _NUM_EXPERTS,), jnp.float32),
            pltpu.VMEM((_CHUNK * _TOP_K,), jnp.int32),
        ],
        compiler_params=pltpu.CompilerParams(needs_layout_passes=False),
    )
    def body(logits_hbm, w_hbm, i_hbm, lbuf, wbuf, ibuf):
        wid = lax.axis_index("s") * _NC + lax.axis_index("c")
        row0 = wid * _ROWS_PER_W

        def chunk(ci, carry):
            r0 = row0 + ci * _CHUNK
            pltpu.sync_copy(
                logits_hbm.at[pl.ds(r0 * _NUM_EXPERTS, _CHUNK * _NUM_EXPERTS)],
                lbuf)
            for sb in range(_CHUNK // _LANES):
                _topk_softmax_subblock(sb, lbuf, wbuf, ibuf)
            pltpu.sync_copy(
                wbuf,
                w_hbm.at[pl.ds(r0 * _NUM_EXPERTS, _CHUNK * _NUM_EXPERTS)])
            pltpu.sync_copy(
                ibuf, i_hbm.at[pl.ds(r0 * _TOP_K, _CHUNK * _TOP_K)])
            return carry

        lax.fori_loop(0, _ROWS_PER_W // _CHUNK, chunk, 0)

    return body(logits_flat)


@jax.jit
def kernel(x, W, b):
    logits = _router_logits(x, W, b)
    w_flat, i_flat = _gating_sc(logits.reshape(-1))
    return (w_flat.reshape(_N_TOKENS, _NUM_EXPERTS),
            i_flat.reshape(_N_TOKENS, _TOP_K))
